# baseline (speedup 1.0000x reference)
import functools

import jax
import jax.numpy as jnp
from jax import lax
from jax.experimental import pallas as pl
from jax.experimental.pallas import tpu as pltpu


def _body(NB, Bn, A, cls_ref, reg_ref, anc_ref, ann_ref, out_ref, acc_ref):
    b = pl.program_id(0)
    i = pl.program_id(1)
    M = ann_ref.shape[1]
    C = cls_ref.shape[1]
    BA = cls_ref.shape[2]

    lane_a = lax.broadcasted_iota(jnp.int32, (1, BA), 1)
    alive = (i * BA + lane_a) < A
    p = jnp.where(alive, cls_ref[0], 0.5)
    ann = ann_ref[0]
    bx1 = ann[:, 0:1]
    by1 = ann[:, 1:2]
    bx2 = ann[:, 2:3]
    by2 = ann[:, 3:4]
    anc = anc_ref[0]
    ax1 = anc[0:1, :]
    ay1 = anc[1:2, :]
    ax2 = anc[2:3, :]
    ay2 = anc[3:4, :]

    iw = jnp.maximum(jnp.minimum(ax2, bx2) - jnp.maximum(ax1, bx1), 0.0)
    ih = jnp.maximum(jnp.minimum(ay2, by2) - jnp.maximum(ay1, by1), 0.0)
    inter = iw * ih
    area_b = (bx2 - bx1) * (by2 - by1)
    area_a = (ax2 - ax1) * (ay2 - ay1)
    ua = jnp.maximum(area_a + area_b - inter, 1e-8)
    iou = inter / ua

    iou_max = jnp.max(iou, axis=0, keepdims=True)
    m_iota = lax.broadcasted_iota(jnp.int32, (M, BA), 0)
    iou_arg = jnp.min(jnp.where(iou == iou_max, m_iota, M), axis=0,
                      keepdims=True)
    onehot = (m_iota == iou_arg).astype(jnp.float32)

    pos = jnp.logical_and(iou_max >= 0.5, alive)
    neg = iou_max < 0.4
    posf = pos.astype(jnp.float32)
    validf = jnp.logical_and(jnp.logical_or(pos, neg), alive).astype(jnp.float32)
    npos_blk = jnp.sum(posf)

    f0n = (p * p) * jnp.log(1.0 - p)
    s0col = lax.dot_general(f0n, validf, (((1,), (1,)), ((), ())),
                            preferred_element_type=jnp.float32)
    cls_blk = -0.75 * jnp.sum(s0col)

    gcx_m = 0.5 * (bx1 + bx2)
    gcy_m = 0.5 * (by1 + by2)
    lgw_m = jnp.log(jnp.maximum(bx2 - bx1, 1.0))
    lgh_m = jnp.log(jnp.maximum(by2 - by1, 1.0))
    G = jnp.concatenate([gcx_m, gcy_m, lgw_m, lgh_m], axis=1)
    assigned = lax.dot_general(G, onehot, (((0,), (0,)), ((), ())),
                               preferred_element_type=jnp.float32)
    lab = ann[:, 4:5]

    c_iota = lax.broadcasted_iota(jnp.int32, (M, C), 1)
    sel = (c_iota == lab.astype(jnp.int32)).astype(jnp.float32)
    pcolsT = lax.dot_general(sel, p, (((1,), (0,)), ((), ())),
                             preferred_element_type=jnp.float32)
    q = jnp.sum(pcolsT * onehot, axis=0, keepdims=True)
    q = jnp.minimum(jnp.maximum(q, 1e-4), 1.0 - 1e-4)
    f0q = (0.75 * q * q) * (-jnp.log(1.0 - q))
    f1q = (0.25 * (1.0 - q) * (1.0 - q)) * (-jnp.log(q))
    cls_blk += jnp.sum(jnp.where(pos, f1q - f0q, 0.0))

    aw = ax2 - ax1
    ah = ay2 - ay1
    aw = jnp.where(aw > 0.0, aw, 1.0)
    ah = jnp.where(ah > 0.0, ah, 1.0)
    acx = ax1 + 0.5 * aw
    acy = ay1 + 0.5 * ah
    inv_aw = 1.0 / aw
    inv_ah = 1.0 / ah
    tdx = ((assigned[0:1, :] - acx) * inv_aw) * 10.0
    tdy = ((assigned[1:2, :] - acy) * inv_ah) * 10.0
    tdw = (assigned[2:3, :] - jnp.log(aw)) * 5.0
    tdh = (assigned[3:4, :] - jnp.log(ah)) * 5.0
    r = reg_ref[0]

    def _sl1(d):
        return jnp.where(d <= 1.0 / 9.0, 4.5 * d * d, d - 1.0 / 18.0)

    rsum = (_sl1(jnp.abs(tdx - r[0:1, :])) + _sl1(jnp.abs(tdy - r[1:2, :]))
            + _sl1(jnp.abs(tdw - r[2:3, :])) + _sl1(jnp.abs(tdh - r[3:4, :])))
    reg_blk = jnp.sum(jnp.where(pos, rsum, 0.0))

    lane = lax.broadcasted_iota(jnp.int32, (1, 128), 1)

    @pl.when(jnp.logical_and(b == 0, i == 0))
    def _init_out():
        out_ref[...] = jnp.zeros_like(out_ref)

    @pl.when(i == 0)
    def _init_acc():
        acc_ref[...] = jnp.zeros_like(acc_ref)

    acc_ref[...] += (jnp.where(lane == 0, cls_blk, 0.0)
                     + jnp.where(lane == 1, reg_blk, 0.0)
                     + jnp.where(lane == 2, npos_blk, 0.0))

    @pl.when(i == NB - 1)
    def _finalize():
        acc = acc_ref[...]
        csum = jnp.sum(jnp.where(lane == 0, acc, 0.0))
        rsum_t = jnp.sum(jnp.where(lane == 1, acc, 0.0))
        npv = jnp.sum(jnp.where(lane == 2, acc, 0.0))
        npc = jnp.maximum(npv, 1.0)
        cl = csum / npc
        rl = jnp.where(npv > 0.0, rsum_t / (npc * 4.0), 0.0)
        out_ref[...] += (jnp.where(lane == 0, cl / Bn, 0.0)
                         + jnp.where(lane == 1, rl / Bn, 0.0))


def kernel(classifications, regressions, anchors, annotations):
    Bn, A, C = classifications.shape
    M = annotations.shape[1]
    BA = 5120 if A >= 5120 else ((A + 127) // 128) * 128
    NB = -(-A // BA)
    cls_t = jnp.swapaxes(classifications, 1, 2)
    reg_t = jnp.swapaxes(regressions, 1, 2)
    anc_t = jnp.swapaxes(anchors, 1, 2)

    out = pl.pallas_call(
        functools.partial(_body, NB, Bn, A),
        grid=(Bn, NB),
        in_specs=[
            pl.BlockSpec((1, C, BA), lambda b, i: (b, 0, i)),
            pl.BlockSpec((1, 4, BA), lambda b, i: (b, 0, i)),
            pl.BlockSpec((1, 4, BA), lambda b, i: (0, 0, i)),
            pl.BlockSpec((1, M, 5), lambda b, i: (b, 0, 0)),
        ],
        out_specs=pl.BlockSpec((1, 128), lambda b, i: (0, 0)),
        out_shape=jax.ShapeDtypeStruct((1, 128), jnp.float32),
        scratch_shapes=[pltpu.VMEM((1, 128), jnp.float32)],
    )(cls_t, reg_t, anc_t, annotations)
    return (out[0, 0:1], out[0, 1:2])


# speedup vs baseline: 1.0889x; 1.0889x over previous
import functools

import jax
import jax.numpy as jnp
from jax import lax
from jax.experimental import pallas as pl
from jax.experimental.pallas import tpu as pltpu


def _body(NB, Bn, A, cls_ref, reg_ref, anc_ref, ann_ref, out_ref, acc_ref):
    b = pl.program_id(0)
    i = pl.program_id(1)
    M = ann_ref.shape[1]
    C = cls_ref.shape[1]
    BA = cls_ref.shape[2]

    lane_a = lax.broadcasted_iota(jnp.int32, (1, BA), 1)
    alive = (i * BA + lane_a) < A
    p = jnp.where(alive, cls_ref[0], 0.5)
    ann = ann_ref[0]
    bx1 = ann[:, 0:1]
    by1 = ann[:, 1:2]
    bx2 = ann[:, 2:3]
    by2 = ann[:, 3:4]
    anc = anc_ref[0]
    ax1 = anc[0:1, :]
    ay1 = anc[1:2, :]
    ax2 = anc[2:3, :]
    ay2 = anc[3:4, :]

    iw = jnp.maximum(jnp.minimum(ax2, bx2) - jnp.maximum(ax1, bx1), 0.0)
    ih = jnp.maximum(jnp.minimum(ay2, by2) - jnp.maximum(ay1, by1), 0.0)
    inter = iw * ih
    area_b = (bx2 - bx1) * (by2 - by1)
    area_a = (ax2 - ax1) * (ay2 - ay1)
    ua = jnp.maximum(area_a + area_b - inter, 1e-8)
    iou = inter / ua

    iou_max = jnp.max(iou, axis=0, keepdims=True)
    m_iota = lax.broadcasted_iota(jnp.int32, (M, BA), 0)
    iou_arg = jnp.min(jnp.where(iou == iou_max, m_iota, M), axis=0,
                      keepdims=True)
    onehot = (m_iota == iou_arg).astype(jnp.float32)

    pos = jnp.logical_and(iou_max >= 0.5, alive)
    neg = iou_max < 0.4
    posf = pos.astype(jnp.float32)
    validf = jnp.logical_and(jnp.logical_or(pos, neg), alive).astype(jnp.float32)
    npos_blk = jnp.sum(posf)

    f0n = (p * p) * jnp.log(1.0 - p)
    s0col = lax.dot_general(f0n, validf, (((1,), (1,)), ((), ())),
                            preferred_element_type=jnp.float32)
    cls_blk = -0.75 * jnp.sum(s0col)

    assigned = lax.dot_general(ann, onehot, (((0,), (0,)), ((), ())),
                               preferred_element_type=jnp.float32)
    gx1 = assigned[0:1, :]
    gy1 = assigned[1:2, :]
    gx2 = assigned[2:3, :]
    gy2 = assigned[3:4, :]
    lab = ann[:, 4:5]

    c_iota = lax.broadcasted_iota(jnp.int32, (M, C), 1)
    sel = (c_iota == lab.astype(jnp.int32)).astype(jnp.float32)
    pcolsT = lax.dot_general(sel, p, (((1,), (0,)), ((), ())),
                             preferred_element_type=jnp.float32)
    q = jnp.sum(pcolsT * onehot, axis=0, keepdims=True)
    q = jnp.minimum(jnp.maximum(q, 1e-4), 1.0 - 1e-4)
    f0q = (0.75 * q * q) * (-jnp.log(1.0 - q))
    f1q = (0.25 * (1.0 - q) * (1.0 - q)) * (-jnp.log(q))
    cls_blk += jnp.sum(jnp.where(pos, f1q - f0q, 0.0))

    aw = ax2 - ax1
    ah = ay2 - ay1
    aw = jnp.where(aw > 0.0, aw, 1.0)
    ah = jnp.where(ah > 0.0, ah, 1.0)
    acx = ax1 + 0.5 * aw
    acy = ay1 + 0.5 * ah
    gwr = gx2 - gx1
    ghr = gy2 - gy1
    gcx = gx1 + 0.5 * gwr
    gcy = gy1 + 0.5 * ghr
    gw = jnp.maximum(gwr, 1.0)
    gh = jnp.maximum(ghr, 1.0)
    inv_aw = 1.0 / aw
    inv_ah = 1.0 / ah
    tdx = ((gcx - acx) * inv_aw) * 10.0
    tdy = ((gcy - acy) * inv_ah) * 10.0
    tdw = jnp.log(gw * inv_aw) * 5.0
    tdh = jnp.log(gh * inv_ah) * 5.0
    r = reg_ref[0]

    def _sl1(d):
        return jnp.where(d <= 1.0 / 9.0, 4.5 * d * d, d - 1.0 / 18.0)

    rsum = (_sl1(jnp.abs(tdx - r[0:1, :])) + _sl1(jnp.abs(tdy - r[1:2, :]))
            + _sl1(jnp.abs(tdw - r[2:3, :])) + _sl1(jnp.abs(tdh - r[3:4, :])))
    reg_blk = jnp.sum(jnp.where(pos, rsum, 0.0))

    lane = lax.broadcasted_iota(jnp.int32, (1, 128), 1)

    @pl.when(jnp.logical_and(b == 0, i == 0))
    def _init_out():
        out_ref[...] = jnp.zeros_like(out_ref)

    @pl.when(i == 0)
    def _init_acc():
        acc_ref[...] = jnp.zeros_like(acc_ref)

    acc_ref[...] += (jnp.where(lane == 0, cls_blk, 0.0)
                     + jnp.where(lane == 1, reg_blk, 0.0)
                     + jnp.where(lane == 2, npos_blk, 0.0))

    @pl.when(i == NB - 1)
    def _finalize():
        acc = acc_ref[...]
        csum = jnp.sum(jnp.where(lane == 0, acc, 0.0))
        rsum_t = jnp.sum(jnp.where(lane == 1, acc, 0.0))
        npv = jnp.sum(jnp.where(lane == 2, acc, 0.0))
        npc = jnp.maximum(npv, 1.0)
        cl = csum / npc
        rl = jnp.where(npv > 0.0, rsum_t / (npc * 4.0), 0.0)
        out_ref[...] += (jnp.where(lane == 0, cl / Bn, 0.0)
                         + jnp.where(lane == 1, rl / Bn, 0.0))


def kernel(classifications, regressions, anchors, annotations):
    Bn, A, C = classifications.shape
    M = annotations.shape[1]
    BA = 6400 if A >= 6400 else ((A + 127) // 128) * 128
    NB = -(-A // BA)
    cls_t = jnp.swapaxes(classifications, 1, 2)
    reg_t = jnp.swapaxes(regressions, 1, 2)
    anc_t = jnp.swapaxes(anchors, 1, 2)

    out = pl.pallas_call(
        functools.partial(_body, NB, Bn, A),
        grid=(Bn, NB),
        in_specs=[
            pl.BlockSpec((1, C, BA), lambda b, i: (b, 0, i)),
            pl.BlockSpec((1, 4, BA), lambda b, i: (b, 0, i)),
            pl.BlockSpec((1, 4, BA), lambda b, i: (0, 0, i)),
            pl.BlockSpec((1, M, 5), lambda b, i: (b, 0, 0)),
        ],
        out_specs=pl.BlockSpec((1, 128), lambda b, i: (0, 0)),
        out_shape=jax.ShapeDtypeStruct((1, 128), jnp.float32),
        scratch_shapes=[pltpu.VMEM((1, 128), jnp.float32)],
    )(cls_t, reg_t, anc_t, annotations)
    return (out[0, 0:1], out[0, 1:2])


# speedup vs baseline: 1.1708x; 1.0752x over previous
import functools

import jax
import jax.numpy as jnp
from jax import lax
from jax.experimental import pallas as pl
from jax.experimental.pallas import tpu as pltpu


def _body(NB, Bn, A, cls_ref, reg_ref, anc_ref, ann_ref, out_ref, acc_ref):
    b = pl.program_id(0)
    i = pl.program_id(1)
    M = ann_ref.shape[1]
    C = cls_ref.shape[1]
    BA = cls_ref.shape[2]

    lane_a = lax.broadcasted_iota(jnp.int32, (1, BA), 1)
    alive = (i * BA + lane_a) < A
    p = jnp.where(alive, cls_ref[0], 0.5)
    ann = ann_ref[0]
    bx1 = ann[:, 0:1]
    by1 = ann[:, 1:2]
    bx2 = ann[:, 2:3]
    by2 = ann[:, 3:4]
    anc = anc_ref[0]
    ax1 = anc[0:1, :]
    ay1 = anc[1:2, :]
    ax2 = anc[2:3, :]
    ay2 = anc[3:4, :]

    iw = jnp.maximum(jnp.minimum(ax2, bx2) - jnp.maximum(ax1, bx1), 0.0)
    ih = jnp.maximum(jnp.minimum(ay2, by2) - jnp.maximum(ay1, by1), 0.0)
    inter = iw * ih
    area_b = (bx2 - bx1) * (by2 - by1)
    area_a = (ax2 - ax1) * (ay2 - ay1)
    ua = jnp.maximum(area_a + area_b - inter, 1e-8)
    iou = inter / ua

    iou_max = jnp.max(iou, axis=0, keepdims=True)
    m_iota = lax.broadcasted_iota(jnp.int32, (M, BA), 0)
    iou_arg = jnp.min(jnp.where(iou == iou_max, m_iota, M), axis=0,
                      keepdims=True)
    onehot = (m_iota == iou_arg).astype(jnp.float32)

    pos = jnp.logical_and(iou_max >= 0.5, alive)
    neg = iou_max < 0.4
    posf = pos.astype(jnp.float32)
    validf = jnp.logical_and(jnp.logical_or(pos, neg), alive).astype(jnp.float32)
    npos_blk = jnp.sum(posf)

    f0n = (p * p) * jnp.log(1.0 - p)
    s0col = lax.dot_general(f0n, validf, (((1,), (1,)), ((), ())),
                            preferred_element_type=jnp.float32)
    cls_blk = -0.75 * jnp.sum(s0col)

    assigned = lax.dot_general(ann, onehot, (((0,), (0,)), ((), ())),
                               preferred_element_type=jnp.float32)
    gx1 = assigned[0:1, :]
    gy1 = assigned[1:2, :]
    gx2 = assigned[2:3, :]
    gy2 = assigned[3:4, :]
    lab = ann[:, 4:5]

    c_iota = lax.broadcasted_iota(jnp.int32, (M, C), 1)
    sel = (c_iota == lab.astype(jnp.int32)).astype(jnp.float32)
    pcolsT = lax.dot_general(sel, p, (((1,), (0,)), ((), ())),
                             preferred_element_type=jnp.float32)
    q = jnp.sum(pcolsT * onehot, axis=0, keepdims=True)
    q = jnp.minimum(jnp.maximum(q, 1e-4), 1.0 - 1e-4)
    f0q = (0.75 * q * q) * (-jnp.log(1.0 - q))
    f1q = (0.25 * (1.0 - q) * (1.0 - q)) * (-jnp.log(q))
    cls_blk += jnp.sum(jnp.where(pos, f1q - f0q, 0.0))

    aw = ax2 - ax1
    ah = ay2 - ay1
    aw = jnp.where(aw > 0.0, aw, 1.0)
    ah = jnp.where(ah > 0.0, ah, 1.0)
    acx = ax1 + 0.5 * aw
    acy = ay1 + 0.5 * ah
    gwr = gx2 - gx1
    ghr = gy2 - gy1
    gcx = gx1 + 0.5 * gwr
    gcy = gy1 + 0.5 * ghr
    gw = jnp.maximum(gwr, 1.0)
    gh = jnp.maximum(ghr, 1.0)
    inv_aw = 1.0 / aw
    inv_ah = 1.0 / ah
    tdx = ((gcx - acx) * inv_aw) * 10.0
    tdy = ((gcy - acy) * inv_ah) * 10.0
    tdw = jnp.log(gw * inv_aw) * 5.0
    tdh = jnp.log(gh * inv_ah) * 5.0
    r = reg_ref[0]

    def _sl1(d):
        return jnp.where(d <= 1.0 / 9.0, 4.5 * d * d, d - 1.0 / 18.0)

    rsum = (_sl1(jnp.abs(tdx - r[0:1, :])) + _sl1(jnp.abs(tdy - r[1:2, :]))
            + _sl1(jnp.abs(tdw - r[2:3, :])) + _sl1(jnp.abs(tdh - r[3:4, :])))
    reg_blk = jnp.sum(jnp.where(pos, rsum, 0.0))

    lane = lax.broadcasted_iota(jnp.int32, (1, 128), 1)

    @pl.when(jnp.logical_and(b == 0, i == 0))
    def _init_out():
        out_ref[...] = jnp.zeros_like(out_ref)

    @pl.when(i == 0)
    def _init_acc():
        acc_ref[...] = jnp.zeros_like(acc_ref)

    acc_ref[...] += (jnp.where(lane == 0, cls_blk, 0.0)
                     + jnp.where(lane == 1, reg_blk, 0.0)
                     + jnp.where(lane == 2, npos_blk, 0.0))

    @pl.when(i == NB - 1)
    def _finalize():
        acc = acc_ref[...]
        csum = jnp.sum(jnp.where(lane == 0, acc, 0.0))
        rsum_t = jnp.sum(jnp.where(lane == 1, acc, 0.0))
        npv = jnp.sum(jnp.where(lane == 2, acc, 0.0))
        npc = jnp.maximum(npv, 1.0)
        cl = csum / npc
        rl = jnp.where(npv > 0.0, rsum_t / (npc * 4.0), 0.0)
        out_ref[...] += (jnp.where(lane == 0, cl / Bn, 0.0)
                         + jnp.where(lane == 1, rl / Bn, 0.0))


def kernel(classifications, regressions, anchors, annotations):
    Bn, A, C = classifications.shape
    M = annotations.shape[1]
    BA = 12800 if A >= 12800 else ((A + 127) // 128) * 128
    NB = -(-A // BA)
    cls_t = jnp.swapaxes(classifications, 1, 2)
    reg_t = jnp.swapaxes(regressions, 1, 2)
    anc_t = jnp.swapaxes(anchors, 1, 2)

    out = pl.pallas_call(
        functools.partial(_body, NB, Bn, A),
        grid=(Bn, NB),
        in_specs=[
            pl.BlockSpec((1, C, BA), lambda b, i: (b, 0, i)),
            pl.BlockSpec((1, 4, BA), lambda b, i: (b, 0, i)),
            pl.BlockSpec((1, 4, BA), lambda b, i: (0, 0, i)),
            pl.BlockSpec((1, M, 5), lambda b, i: (b, 0, 0)),
        ],
        out_specs=pl.BlockSpec((1, 128), lambda b, i: (0, 0)),
        out_shape=jax.ShapeDtypeStruct((1, 128), jnp.float32),
        scratch_shapes=[pltpu.VMEM((1, 128), jnp.float32)],
    )(cls_t, reg_t, anc_t, annotations)
    return (out[0, 0:1], out[0, 1:2])


# speedup vs baseline: 1.2145x; 1.0373x over previous
import functools

import jax
import jax.numpy as jnp
from jax import lax
from jax.experimental import pallas as pl
from jax.experimental.pallas import tpu as pltpu


def _body(NB, Bn, A, cls_ref, reg_ref, anc_ref, ann_ref, out_ref, acc_ref):
    b = pl.program_id(0)
    i = pl.program_id(1)
    M = ann_ref.shape[1]
    C = cls_ref.shape[1]
    BA = cls_ref.shape[2]

    lane_a = lax.broadcasted_iota(jnp.int32, (1, BA), 1)
    alive = (i * BA + lane_a) < A
    p = jnp.where(alive, cls_ref[0], 0.5)
    ann = ann_ref[0]
    bx1 = ann[:, 0:1]
    by1 = ann[:, 1:2]
    bx2 = ann[:, 2:3]
    by2 = ann[:, 3:4]
    anc = anc_ref[0]
    ax1 = anc[0:1, :]
    ay1 = anc[1:2, :]
    ax2 = anc[2:3, :]
    ay2 = anc[3:4, :]

    iw = jnp.maximum(jnp.minimum(ax2, bx2) - jnp.maximum(ax1, bx1), 0.0)
    ih = jnp.maximum(jnp.minimum(ay2, by2) - jnp.maximum(ay1, by1), 0.0)
    inter = iw * ih
    area_b = (bx2 - bx1) * (by2 - by1)
    area_a = (ax2 - ax1) * (ay2 - ay1)
    ua = jnp.maximum(area_a + area_b - inter, 1e-8)
    iou = inter / ua

    iou_max = jnp.max(iou, axis=0, keepdims=True)
    m_iota = lax.broadcasted_iota(jnp.int32, (M, BA), 0)
    iou_arg = jnp.min(jnp.where(iou == iou_max, m_iota, M), axis=0,
                      keepdims=True)
    onehot = (m_iota == iou_arg).astype(jnp.float32)

    pos = jnp.logical_and(iou_max >= 0.5, alive)
    neg = iou_max < 0.4
    posf = pos.astype(jnp.float32)
    validf = jnp.logical_and(jnp.logical_or(pos, neg), alive).astype(jnp.float32)
    npos_blk = jnp.sum(posf)

    f0n = (p * p) * jnp.log(1.0 - p)
    s0col = lax.dot_general(f0n, validf, (((1,), (1,)), ((), ())),
                            preferred_element_type=jnp.float32)
    cls_blk = -0.75 * jnp.sum(s0col)

    assigned = lax.dot_general(ann, onehot, (((0,), (0,)), ((), ())),
                               preferred_element_type=jnp.float32)
    gx1 = assigned[0:1, :]
    gy1 = assigned[1:2, :]
    gx2 = assigned[2:3, :]
    gy2 = assigned[3:4, :]
    lab = ann[:, 4:5]

    c_iota = lax.broadcasted_iota(jnp.int32, (M, C), 1)
    sel = (c_iota == lab.astype(jnp.int32)).astype(jnp.float32)
    pcolsT = lax.dot_general(sel, p, (((1,), (0,)), ((), ())),
                             preferred_element_type=jnp.float32)
    q = jnp.sum(pcolsT * onehot, axis=0, keepdims=True)
    q = jnp.minimum(jnp.maximum(q, 1e-4), 1.0 - 1e-4)
    f0q = (0.75 * q * q) * (-jnp.log(1.0 - q))
    f1q = (0.25 * (1.0 - q) * (1.0 - q)) * (-jnp.log(q))
    cls_blk += jnp.sum(jnp.where(pos, f1q - f0q, 0.0))

    aw = ax2 - ax1
    ah = ay2 - ay1
    aw = jnp.where(aw > 0.0, aw, 1.0)
    ah = jnp.where(ah > 0.0, ah, 1.0)
    acx = ax1 + 0.5 * aw
    acy = ay1 + 0.5 * ah
    gwr = gx2 - gx1
    ghr = gy2 - gy1
    gcx = gx1 + 0.5 * gwr
    gcy = gy1 + 0.5 * ghr
    gw = jnp.maximum(gwr, 1.0)
    gh = jnp.maximum(ghr, 1.0)
    inv_aw = 1.0 / aw
    inv_ah = 1.0 / ah
    tdx = ((gcx - acx) * inv_aw) * 10.0
    tdy = ((gcy - acy) * inv_ah) * 10.0
    tdw = jnp.log(gw * inv_aw) * 5.0
    tdh = jnp.log(gh * inv_ah) * 5.0
    r = reg_ref[0]

    def _sl1(d):
        return jnp.where(d <= 1.0 / 9.0, 4.5 * d * d, d - 1.0 / 18.0)

    rsum = (_sl1(jnp.abs(tdx - r[0:1, :])) + _sl1(jnp.abs(tdy - r[1:2, :]))
            + _sl1(jnp.abs(tdw - r[2:3, :])) + _sl1(jnp.abs(tdh - r[3:4, :])))
    reg_blk = jnp.sum(jnp.where(pos, rsum, 0.0))

    lane = lax.broadcasted_iota(jnp.int32, (1, 128), 1)

    @pl.when(jnp.logical_and(b == 0, i == 0))
    def _init_out():
        out_ref[...] = jnp.zeros_like(out_ref)

    @pl.when(i == 0)
    def _init_acc():
        acc_ref[...] = jnp.zeros_like(acc_ref)

    acc_ref[...] += (jnp.where(lane == 0, cls_blk, 0.0)
                     + jnp.where(lane == 1, reg_blk, 0.0)
                     + jnp.where(lane == 2, npos_blk, 0.0))

    @pl.when(i == NB - 1)
    def _finalize():
        acc = acc_ref[...]
        csum = jnp.sum(jnp.where(lane == 0, acc, 0.0))
        rsum_t = jnp.sum(jnp.where(lane == 1, acc, 0.0))
        npv = jnp.sum(jnp.where(lane == 2, acc, 0.0))
        npc = jnp.maximum(npv, 1.0)
        cl = csum / npc
        rl = jnp.where(npv > 0.0, rsum_t / (npc * 4.0), 0.0)
        out_ref[...] += (jnp.where(lane == 0, cl / Bn, 0.0)
                         + jnp.where(lane == 1, rl / Bn, 0.0))


def kernel(classifications, regressions, anchors, annotations):
    Bn, A, C = classifications.shape
    M = annotations.shape[1]
    BA = 25600 if A >= 25600 else ((A + 127) // 128) * 128
    NB = -(-A // BA)
    cls_t = jnp.swapaxes(classifications, 1, 2)
    reg_t = jnp.swapaxes(regressions, 1, 2)
    anc_t = jnp.swapaxes(anchors, 1, 2)

    out = pl.pallas_call(
        functools.partial(_body, NB, Bn, A),
        grid=(Bn, NB),
        in_specs=[
            pl.BlockSpec((1, C, BA), lambda b, i: (b, 0, i)),
            pl.BlockSpec((1, 4, BA), lambda b, i: (b, 0, i)),
            pl.BlockSpec((1, 4, BA), lambda b, i: (0, 0, i)),
            pl.BlockSpec((1, M, 5), lambda b, i: (b, 0, 0)),
        ],
        out_specs=pl.BlockSpec((1, 128), lambda b, i: (0, 0)),
        out_shape=jax.ShapeDtypeStruct((1, 128), jnp.float32),
        scratch_shapes=[pltpu.VMEM((1, 128), jnp.float32)],
    )(cls_t, reg_t, anc_t, annotations)
    return (out[0, 0:1], out[0, 1:2])


# speedup vs baseline: 1.3740x; 1.1313x over previous
import functools

import jax
import jax.numpy as jnp
from jax import lax
from jax.experimental import pallas as pl
from jax.experimental.pallas import tpu as pltpu


def _body(NB, Bn, A, cls_ref, reg_ref, anc_ref, ann_ref, out_ref, acc_ref):
    b = pl.program_id(0)
    i = pl.program_id(1)
    M = ann_ref.shape[1]
    C = cls_ref.shape[1]
    BA = cls_ref.shape[2]

    lane_a = lax.broadcasted_iota(jnp.int32, (1, BA), 1)
    alive = (i * BA + lane_a) < A
    p = jnp.where(alive, cls_ref[0], 0.5)
    ann = ann_ref[0]
    bx1 = ann[:, 0:1]
    by1 = ann[:, 1:2]
    bx2 = ann[:, 2:3]
    by2 = ann[:, 3:4]
    anc = anc_ref[0]
    ax1 = anc[0:1, :]
    ay1 = anc[1:2, :]
    ax2 = anc[2:3, :]
    ay2 = anc[3:4, :]

    iw = jnp.maximum(jnp.minimum(ax2, bx2) - jnp.maximum(ax1, bx1), 0.0)
    ih = jnp.maximum(jnp.minimum(ay2, by2) - jnp.maximum(ay1, by1), 0.0)
    inter = iw * ih
    area_b = (bx2 - bx1) * (by2 - by1)
    area_a = (ax2 - ax1) * (ay2 - ay1)
    ua = jnp.maximum(area_a + area_b - inter, 1e-8)
    iou = inter / ua

    iou_max = jnp.max(iou, axis=0, keepdims=True)
    m_iota = lax.broadcasted_iota(jnp.int32, (M, BA), 0)
    iou_arg = jnp.min(jnp.where(iou == iou_max, m_iota, M), axis=0,
                      keepdims=True)
    onehot = (m_iota == iou_arg).astype(jnp.float32)

    pos = jnp.logical_and(iou_max >= 0.5, alive)
    neg = iou_max < 0.4
    posf = pos.astype(jnp.float32)
    validf = jnp.logical_and(jnp.logical_or(pos, neg), alive).astype(jnp.float32)
    npos_blk = jnp.sum(posf)

    f0n = (p * p) * jnp.log(1.0 - p)
    s0col = lax.dot_general(f0n, validf, (((1,), (1,)), ((), ())),
                            preferred_element_type=jnp.float32)
    cls_blk = -0.75 * jnp.sum(s0col)

    assigned = lax.dot_general(ann, onehot, (((0,), (0,)), ((), ())),
                               preferred_element_type=jnp.float32)
    gx1 = assigned[0:1, :]
    gy1 = assigned[1:2, :]
    gx2 = assigned[2:3, :]
    gy2 = assigned[3:4, :]
    lab = ann[:, 4:5]

    c_iota = lax.broadcasted_iota(jnp.int32, (M, C), 1)
    sel = (c_iota == lab.astype(jnp.int32)).astype(jnp.float32)
    pcolsT = lax.dot_general(sel, p, (((1,), (0,)), ((), ())),
                             preferred_element_type=jnp.float32)
    q = jnp.sum(pcolsT * onehot, axis=0, keepdims=True)
    q = jnp.minimum(jnp.maximum(q, 1e-4), 1.0 - 1e-4)
    f0q = (0.75 * q * q) * (-jnp.log(1.0 - q))
    f1q = (0.25 * (1.0 - q) * (1.0 - q)) * (-jnp.log(q))
    cls_blk += jnp.sum(jnp.where(pos, f1q - f0q, 0.0))

    aw = ax2 - ax1
    ah = ay2 - ay1
    aw = jnp.where(aw > 0.0, aw, 1.0)
    ah = jnp.where(ah > 0.0, ah, 1.0)
    acx = ax1 + 0.5 * aw
    acy = ay1 + 0.5 * ah
    gwr = gx2 - gx1
    ghr = gy2 - gy1
    gcx = gx1 + 0.5 * gwr
    gcy = gy1 + 0.5 * ghr
    gw = jnp.maximum(gwr, 1.0)
    gh = jnp.maximum(ghr, 1.0)
    inv_aw = 1.0 / aw
    inv_ah = 1.0 / ah
    tdx = ((gcx - acx) * inv_aw) * 10.0
    tdy = ((gcy - acy) * inv_ah) * 10.0
    tdw = jnp.log(gw * inv_aw) * 5.0
    tdh = jnp.log(gh * inv_ah) * 5.0
    r = reg_ref[0]

    def _sl1(d):
        return jnp.where(d <= 1.0 / 9.0, 4.5 * d * d, d - 1.0 / 18.0)

    rsum = (_sl1(jnp.abs(tdx - r[0:1, :])) + _sl1(jnp.abs(tdy - r[1:2, :]))
            + _sl1(jnp.abs(tdw - r[2:3, :])) + _sl1(jnp.abs(tdh - r[3:4, :])))
    reg_blk = jnp.sum(jnp.where(pos, rsum, 0.0))

    lane = lax.broadcasted_iota(jnp.int32, (1, 128), 1)

    @pl.when(jnp.logical_and(b == 0, i == 0))
    def _init_out():
        out_ref[...] = jnp.zeros_like(out_ref)

    @pl.when(i == 0)
    def _init_acc():
        acc_ref[...] = jnp.zeros_like(acc_ref)

    acc_ref[...] += (jnp.where(lane == 0, cls_blk, 0.0)
                     + jnp.where(lane == 1, reg_blk, 0.0)
                     + jnp.where(lane == 2, npos_blk, 0.0))

    @pl.when(i == NB - 1)
    def _finalize():
        acc = acc_ref[...]
        csum = jnp.sum(jnp.where(lane == 0, acc, 0.0))
        rsum_t = jnp.sum(jnp.where(lane == 1, acc, 0.0))
        npv = jnp.sum(jnp.where(lane == 2, acc, 0.0))
        npc = jnp.maximum(npv, 1.0)
        cl = csum / npc
        rl = jnp.where(npv > 0.0, rsum_t / (npc * 4.0), 0.0)
        out_ref[...] += (jnp.where(lane == 0, cl / Bn, 0.0)
                         + jnp.where(lane == 1, rl / Bn, 0.0))


def kernel(classifications, regressions, anchors, annotations):
    Bn, A, C = classifications.shape
    M = annotations.shape[1]
    BA = 51200 if A >= 51200 else ((A + 127) // 128) * 128
    NB = -(-A // BA)
    cls_t = jnp.swapaxes(classifications, 1, 2)
    reg_t = jnp.swapaxes(regressions, 1, 2)
    anc_t = jnp.swapaxes(anchors, 1, 2)

    out = pl.pallas_call(
        functools.partial(_body, NB, Bn, A),
        grid=(Bn, NB),
        in_specs=[
            pl.BlockSpec((1, C, BA), lambda b, i: (b, 0, i)),
            pl.BlockSpec((1, 4, BA), lambda b, i: (b, 0, i)),
            pl.BlockSpec((1, 4, BA), lambda b, i: (0, 0, i)),
            pl.BlockSpec((1, M, 5), lambda b, i: (b, 0, 0)),
        ],
        out_specs=pl.BlockSpec((1, 128), lambda b, i: (0, 0)),
        out_shape=jax.ShapeDtypeStruct((1, 128), jnp.float32),
        scratch_shapes=[pltpu.VMEM((1, 128), jnp.float32)],
    )(cls_t, reg_t, anc_t, annotations)
    return (out[0, 0:1], out[0, 1:2])
